# baseline (speedup 1.0000x reference)
import functools

import jax
import jax.numpy as jnp
from jax import lax
from jax.experimental import pallas as pl
from jax.experimental.pallas import tpu as pltpu
from jax.experimental.pallas import tpu_sc as plsc

N = 10000
E = 160000
G = 64
IN_CH = 256
H = 512
NACC = 10240
EPAD = 163840
CHUNKS = 80
B = 128
NP = 4
PW = H // NP
ROWS_PER_TILE_N = 624
ROWS_PER_TILE_A = NACC // 16
_K = 1
DT = jnp.float32

_MESH = dict(core_axis_name="c", subcore_axis_name="s")


def _deg_body(dst_i, ones_hbm, dpart, dst_v, ones_v, acc):
    c = lax.axis_index("c")
    s = lax.axis_index("s")
    pltpu.sync_copy(dst_i.at[s], dst_v)
    pltpu.sync_copy(ones_hbm.at[pl.ds(0, B)], ones_v)
    pltpu.sync_copy(ones_hbm.at[pl.ds(s * ROWS_PER_TILE_A, ROWS_PER_TILE_A)],
                    acc.at[pl.ds(s * ROWS_PER_TILE_A, ROWS_PER_TILE_A)])
    plsc.subcore_barrier()

    def body(jj, carry):
        j = c * (CHUNKS // 2) + jj
        pltpu.sync_copy(ones_v, acc.at[dst_v.at[j]], add=True)
        return carry

    lax.fori_loop(0, CHUNKS // 2, body, 0)
    plsc.subcore_barrier()
    pltpu.sync_copy(acc.at[pl.ds(s * ROWS_PER_TILE_A, ROWS_PER_TILE_A)],
                    dpart.at[c, pl.ds(s * ROWS_PER_TILE_A, ROWS_PER_TILE_A)])


def _deg_sc(dst_i, ones_hbm):
    f = functools.partial(
        pl.kernel,
        out_type=jax.ShapeDtypeStruct((2, NACC, 8), jnp.float32),
        mesh=plsc.VectorSubcoreMesh(**_MESH),
        scratch_types=[
            pltpu.VMEM((CHUNKS, B), jnp.int32),
            pltpu.VMEM((B, 8), jnp.float32),
            pltpu.VMEM_SHARED((NACC, 8), jnp.float32),
        ],
    )(_deg_body)
    return f(dst_i, ones_hbm)


def _panel_work(s, u_ref, o_ref, src_v, dst_v, rows, gsems, ssems, acc):
    pltpu.sync_copy(u_ref.at[pl.ds(s * ROWS_PER_TILE_N, ROWS_PER_TILE_N)],
                    acc.at[pl.ds(s * ROWS_PER_TILE_N, ROWS_PER_TILE_N)])

    @pl.when(s == 0)
    def _():
        rem = 16 * ROWS_PER_TILE_N
        pltpu.sync_copy(u_ref.at[pl.ds(rem, N - rem)],
                        acc.at[pl.ds(rem, N - rem)])

    plsc.subcore_barrier()

    def body(ii, carry):
        j = ii * _K
        gc = [pltpu.async_copy(u_ref.at[src_v.at[j + k]], rows[k],
                               gsems[k]) for k in range(_K)]
        sc = []
        for k in range(_K):
            gc[k].wait()
            sc.append(pltpu.async_copy(rows[k], acc.at[dst_v.at[j + k]],
                                       ssems[k], add=True))
        for k in range(_K):
            sc[k].wait()
        return carry

    lax.fori_loop(0, CHUNKS // _K, body, 0)
    plsc.subcore_barrier()
    pltpu.sync_copy(acc.at[pl.ds(s * ROWS_PER_TILE_A, ROWS_PER_TILE_A)],
                    o_ref.at[pl.ds(s * ROWS_PER_TILE_A, ROWS_PER_TILE_A)])
    plsc.subcore_barrier()


def _agg_body(*refs):
    us = refs[:NP]
    src_i, dst_i = refs[NP], refs[NP + 1]
    os_ = refs[NP + 2:2 * NP + 2]
    src_v, dst_v = refs[2 * NP + 2], refs[2 * NP + 3]
    rows = refs[2 * NP + 4:2 * NP + 4 + _K]
    acc = refs[2 * NP + 4 + _K]
    sems = refs[2 * NP + 5 + _K:]
    gsems, ssems = sems[:_K], sems[_K:]
    c = lax.axis_index("c")
    s = lax.axis_index("s")
    pltpu.sync_copy(src_i.at[s], src_v)
    pltpu.sync_copy(dst_i.at[s], dst_v)
    half = NP // 2

    @pl.when(c == 0)
    def _():
        for p in range(half):
            _panel_work(s, us[p], os_[p], src_v, dst_v, rows, gsems, ssems,
                        acc)

    @pl.when(c == 1)
    def _():
        for p in range(half, NP):
            _panel_work(s, us[p], os_[p], src_v, dst_v, rows, gsems, ssems,
                        acc)


def _agg_sc(u_panels, src_i, dst_i):
    f = functools.partial(
        pl.kernel,
        out_type=[jax.ShapeDtypeStruct((NACC, PW), DT)] * NP,
        mesh=plsc.VectorSubcoreMesh(**_MESH),
        scratch_types=[
            pltpu.VMEM((CHUNKS, B), jnp.int32),
            pltpu.VMEM((CHUNKS, B), jnp.int32),
        ] + [pltpu.VMEM((B, PW), DT)] * _K + [
            pltpu.VMEM_SHARED((NACC, PW), DT),
        ] + [pltpu.SemaphoreType.DMA] * (2 * _K),
    )(_agg_body)
    return f(*u_panels, src_i, dst_i)


_NB = 2000


def _dinv_of(d_ref):
    deg = d_ref[0, :, 0:1] + d_ref[1, :, 0:1] - 1.0
    return lax.rsqrt(deg)


def _split_store(u, outs):
    ub = u.astype(DT)
    for k, o in enumerate(outs):
        o[...] = ub[:, k * PW:(k + 1) * PW]


def _mm_first_body(x_ref, w_ref, d_ref, *outs):
    dinv = _dinv_of(d_ref)
    xw = jnp.dot(x_ref[...], w_ref[...], preferred_element_type=jnp.float32)
    _split_store(dinv * xw, outs)


def _mm_first(x, w, dpart):
    return pl.pallas_call(
        _mm_first_body,
        grid=(N // _NB,),
        in_specs=[
            pl.BlockSpec((_NB, IN_CH), lambda i: (i, 0)),
            pl.BlockSpec((IN_CH, H), lambda i: (0, 0)),
            pl.BlockSpec((2, _NB, 8), lambda i: (0, i, 0)),
        ],
        out_specs=[pl.BlockSpec((_NB, PW), lambda i: (i, 0))] * NP,
        out_shape=[jax.ShapeDtypeStruct((N, PW), DT)] * NP,
    )(x, w, dpart)


def _relu_concat(os_, dinv, b_ref):
    return jnp.concatenate(
        [jnp.maximum(dinv * o[...].astype(jnp.float32)
                     + b_ref[:, k * PW:(k + 1) * PW], 0.0)
         for k, o in enumerate(os_)], axis=1)


def _mm_mid_body(*refs):
    os_ = refs[:NP]
    d_ref, b_ref, w_ref = refs[NP], refs[NP + 1], refs[NP + 2]
    outs = refs[NP + 3:]
    dinv = _dinv_of(d_ref)
    h = _relu_concat(os_, dinv, b_ref)
    u = dinv * jnp.dot(h, w_ref[...], preferred_element_type=jnp.float32)
    _split_store(u, outs)


def _mm_mid(o_panels, dpart, b_prev, w):
    return pl.pallas_call(
        _mm_mid_body,
        grid=(N // _NB,),
        in_specs=[pl.BlockSpec((_NB, PW), lambda i: (i, 0))] * NP + [
            pl.BlockSpec((2, _NB, 8), lambda i: (0, i, 0)),
            pl.BlockSpec((1, H), lambda i: (0, 0)),
            pl.BlockSpec((H, H), lambda i: (0, 0)),
        ],
        out_specs=[pl.BlockSpec((_NB, PW), lambda i: (i, 0))] * NP,
        out_shape=[jax.ShapeDtypeStruct((N, PW), DT)] * NP,
    )(*o_panels, dpart, b_prev, w)


def _head_body(*refs):
    os_ = refs[:NP]
    (d_ref, b_ref, batch_ref, wl1_ref, bl1_ref, wl2_ref, bl2_ref,
     out_ref, pooled_acc, cnt_acc) = refs[NP:]
    i = pl.program_id(0)

    @pl.when(i == 0)
    def _():
        pooled_acc[...] = jnp.zeros_like(pooled_acc)
        cnt_acc[...] = jnp.zeros_like(cnt_acc)
        out_ref[...] = jnp.zeros_like(out_ref)

    dinv = _dinv_of(d_ref)
    h = _relu_concat(os_, dinv, b_ref)
    seg = lax.broadcasted_iota(jnp.int32, (G, _NB), 0)
    mask = (seg == batch_ref[0, 0, :][None, :]).astype(jnp.float32)
    pooled_acc[...] += jnp.dot(mask, h, preferred_element_type=jnp.float32)
    cnt_acc[...] += jnp.broadcast_to(
        jnp.sum(mask, axis=1, keepdims=True), (G, 128))

    @pl.when(i == pl.num_programs(0) - 1)
    def _():
        cnt = cnt_acc[:, 0:1]
        pooled = pooled_acc[...] / jnp.maximum(cnt, 1.0)
        z = jnp.maximum(
            jnp.dot(pooled, wl1_ref[...], preferred_element_type=jnp.float32)
            + bl1_ref[...], 0.0)
        out_ref[...] = (
            jnp.dot(z, wl2_ref[...], preferred_element_type=jnp.float32)
            + bl2_ref[...])


def _head(o_panels, dpart, b3, batch3d, wl1, bl1, wl2p, bl2p):
    return pl.pallas_call(
        _head_body,
        grid=(N // _NB,),
        in_specs=[pl.BlockSpec((_NB, PW), lambda i: (i, 0))] * NP + [
            pl.BlockSpec((2, _NB, 8), lambda i: (0, i, 0)),
            pl.BlockSpec((1, H), lambda i: (0, 0)),
            pl.BlockSpec((1, 1, _NB), lambda i: (i, 0, 0)),
            pl.BlockSpec((H, H // 2), lambda i: (0, 0)),
            pl.BlockSpec((1, H // 2), lambda i: (0, 0)),
            pl.BlockSpec((H // 2, 128), lambda i: (0, 0)),
            pl.BlockSpec((1, 128), lambda i: (0, 0)),
        ],
        out_specs=pl.BlockSpec((G, 128), lambda i: (0, 0)),
        out_shape=jax.ShapeDtypeStruct((G, 128), jnp.float32),
        scratch_shapes=[
            pltpu.VMEM((G, H), jnp.float32),
            pltpu.VMEM((G, 128), jnp.float32),
        ],
    )(*o_panels, dpart, b3, batch3d, wl1, bl1, wl2p, bl2p)


def kernel(x, edge_index, batch, W1, b1, W2, b2, W3, b3, Wl1, bl1, Wl2, bl2):
    pad = EPAD - E
    src_pad = jnp.concatenate(
        [edge_index[0], jnp.zeros((pad,), jnp.int32)]).reshape(16, CHUNKS, B)
    dst_pad = jnp.concatenate(
        [edge_index[1], jnp.full((pad,), N, jnp.int32)]).reshape(16, CHUNKS, B)
    ones_hbm = jnp.ones((NACC, 8), jnp.float32)

    dpart = _deg_sc(dst_pad, ones_hbm)

    u = _mm_first(x, W1, dpart)
    o = _agg_sc(u, src_pad, dst_pad)
    u = _mm_mid(o, dpart, b1.reshape(1, H), W2)
    o = _agg_sc(u, src_pad, dst_pad)
    u = _mm_mid(o, dpart, b2.reshape(1, H), W3)
    o = _agg_sc(u, src_pad, dst_pad)

    out = _head(o, dpart, b3.reshape(1, H),
                batch.reshape(N // _NB, 1, _NB),
                Wl1, bl1.reshape(1, H // 2),
                jnp.pad(Wl2, ((0, 0), (0, 127))),
                jnp.broadcast_to(bl2.reshape(1, 1), (1, 128)))
    return out[:, 0]


# speedup vs baseline: 1.0494x; 1.0494x over previous
import functools

import jax
import jax.numpy as jnp
from jax import lax
from jax.experimental import pallas as pl
from jax.experimental.pallas import tpu as pltpu
from jax.experimental.pallas import tpu_sc as plsc

N = 10000
E = 160000
G = 64
IN_CH = 256
H = 512
NACC = 10240
EPAD = 163840
CHUNKS = 80
B = 128
NP = 4
PW = H // NP
ROWS_PER_TILE_N = 624
ROWS_PER_TILE_A = NACC // 16
_K = 2
HC = CHUNKS // 2
DT = jnp.float32

_MESH = dict(core_axis_name="c", subcore_axis_name="s")


def _deg_body(dst_i, ones_hbm, dpart, dst_v, ones_v, acc):
    c = lax.axis_index("c")
    s = lax.axis_index("s")
    pltpu.sync_copy(dst_i.at[s], dst_v)
    pltpu.sync_copy(ones_hbm.at[pl.ds(0, B)], ones_v)
    pltpu.sync_copy(ones_hbm.at[pl.ds(s * ROWS_PER_TILE_A, ROWS_PER_TILE_A)],
                    acc.at[pl.ds(s * ROWS_PER_TILE_A, ROWS_PER_TILE_A)])
    plsc.subcore_barrier()

    def body(jj, carry):
        j = c * (CHUNKS // 2) + jj
        pltpu.sync_copy(ones_v, acc.at[dst_v.at[j]], add=True)
        return carry

    lax.fori_loop(0, CHUNKS // 2, body, 0)
    plsc.subcore_barrier()
    pltpu.sync_copy(acc.at[pl.ds(s * ROWS_PER_TILE_A, ROWS_PER_TILE_A)],
                    dpart.at[c, pl.ds(s * ROWS_PER_TILE_A, ROWS_PER_TILE_A)])


def _deg_sc(dst_i, ones_hbm):
    f = functools.partial(
        pl.kernel,
        out_type=jax.ShapeDtypeStruct((2, NACC, 8), jnp.float32),
        mesh=plsc.VectorSubcoreMesh(**_MESH),
        scratch_types=[
            pltpu.VMEM((CHUNKS, B), jnp.int32),
            pltpu.VMEM((B, 8), jnp.float32),
            pltpu.VMEM_SHARED((NACC, 8), jnp.float32),
        ],
    )(_deg_body)
    return f(dst_i, ones_hbm)


def _panel_work(s, u_ref, o_ref, src_i, dst_i, src_v, dst_v, rows, gsems,
                ssems, acc):
    pltpu.sync_copy(u_ref.at[pl.ds(s * ROWS_PER_TILE_N, ROWS_PER_TILE_N)],
                    acc.at[pl.ds(s * ROWS_PER_TILE_N, ROWS_PER_TILE_N)])

    @pl.when(s == 0)
    def _():
        rem = 16 * ROWS_PER_TILE_N
        pltpu.sync_copy(u_ref.at[pl.ds(rem, N - rem)],
                        acc.at[pl.ds(rem, N - rem)])

    plsc.subcore_barrier()

    def body(ii, carry):
        j = ii * _K
        gc = [pltpu.async_copy(u_ref.at[src_v.at[j + k]], rows[k],
                               gsems[k]) for k in range(_K)]
        sc = []
        for k in range(_K):
            gc[k].wait()
            sc.append(pltpu.async_copy(rows[k], acc.at[dst_v.at[j + k]],
                                       ssems[k], add=True))
        for k in range(_K):
            sc[k].wait()
        return carry

    for h in range(CHUNKS // HC):
        pltpu.sync_copy(src_i.at[s, pl.ds(h * HC, HC)], src_v)
        pltpu.sync_copy(dst_i.at[s, pl.ds(h * HC, HC)], dst_v)
        lax.fori_loop(0, HC // _K, body, 0)
    plsc.subcore_barrier()
    pltpu.sync_copy(acc.at[pl.ds(s * ROWS_PER_TILE_A, ROWS_PER_TILE_A)],
                    o_ref.at[pl.ds(s * ROWS_PER_TILE_A, ROWS_PER_TILE_A)])
    plsc.subcore_barrier()


def _agg_body(*refs):
    us = refs[:NP]
    src_i, dst_i = refs[NP], refs[NP + 1]
    os_ = refs[NP + 2:2 * NP + 2]
    src_v, dst_v = refs[2 * NP + 2], refs[2 * NP + 3]
    rows = refs[2 * NP + 4:2 * NP + 4 + _K]
    acc = refs[2 * NP + 4 + _K]
    sems = refs[2 * NP + 5 + _K:]
    gsems, ssems = sems[:_K], sems[_K:]
    c = lax.axis_index("c")
    s = lax.axis_index("s")
    half = NP // 2

    @pl.when(c == 0)
    def _():
        for p in range(half):
            _panel_work(s, us[p], os_[p], src_i, dst_i, src_v, dst_v, rows,
                        gsems, ssems, acc)

    @pl.when(c == 1)
    def _():
        for p in range(half, NP):
            _panel_work(s, us[p], os_[p], src_i, dst_i, src_v, dst_v, rows,
                        gsems, ssems, acc)


def _agg_sc(u_panels, src_i, dst_i):
    f = functools.partial(
        pl.kernel,
        out_type=[jax.ShapeDtypeStruct((NACC, PW), DT)] * NP,
        mesh=plsc.VectorSubcoreMesh(**_MESH),
        scratch_types=[
            pltpu.VMEM((HC, B), jnp.int32),
            pltpu.VMEM((HC, B), jnp.int32),
        ] + [pltpu.VMEM((B, PW), DT)] * _K + [
            pltpu.VMEM_SHARED((NACC, PW), DT),
        ] + [pltpu.SemaphoreType.DMA] * (2 * _K),
    )(_agg_body)
    return f(*u_panels, src_i, dst_i)


_NB = 2000


def _dinv_of(d_ref):
    deg = d_ref[0, :, 0:1] + d_ref[1, :, 0:1] - 1.0
    return lax.rsqrt(deg)


def _split_store(u, outs):
    ub = u.astype(DT)
    for k, o in enumerate(outs):
        o[...] = ub[:, k * PW:(k + 1) * PW]


def _mm_first_body(x_ref, w_ref, d_ref, *outs):
    dinv = _dinv_of(d_ref)
    xw = jnp.dot(x_ref[...], w_ref[...], preferred_element_type=jnp.float32)
    _split_store(dinv * xw, outs)


def _mm_first(x, w, dpart):
    return pl.pallas_call(
        _mm_first_body,
        grid=(N // _NB,),
        in_specs=[
            pl.BlockSpec((_NB, IN_CH), lambda i: (i, 0)),
            pl.BlockSpec((IN_CH, H), lambda i: (0, 0)),
            pl.BlockSpec((2, _NB, 8), lambda i: (0, i, 0)),
        ],
        out_specs=[pl.BlockSpec((_NB, PW), lambda i: (i, 0))] * NP,
        out_shape=[jax.ShapeDtypeStruct((N, PW), DT)] * NP,
    )(x, w, dpart)


def _relu_concat(os_, dinv, b_ref):
    return jnp.concatenate(
        [jnp.maximum(dinv * o[...].astype(jnp.float32)
                     + b_ref[:, k * PW:(k + 1) * PW], 0.0)
         for k, o in enumerate(os_)], axis=1)


def _mm_mid_body(*refs):
    os_ = refs[:NP]
    d_ref, b_ref, w_ref = refs[NP], refs[NP + 1], refs[NP + 2]
    outs = refs[NP + 3:]
    dinv = _dinv_of(d_ref)
    h = _relu_concat(os_, dinv, b_ref)
    u = dinv * jnp.dot(h, w_ref[...], preferred_element_type=jnp.float32)
    _split_store(u, outs)


def _mm_mid(o_panels, dpart, b_prev, w):
    return pl.pallas_call(
        _mm_mid_body,
        grid=(N // _NB,),
        in_specs=[pl.BlockSpec((_NB, PW), lambda i: (i, 0))] * NP + [
            pl.BlockSpec((2, _NB, 8), lambda i: (0, i, 0)),
            pl.BlockSpec((1, H), lambda i: (0, 0)),
            pl.BlockSpec((H, H), lambda i: (0, 0)),
        ],
        out_specs=[pl.BlockSpec((_NB, PW), lambda i: (i, 0))] * NP,
        out_shape=[jax.ShapeDtypeStruct((N, PW), DT)] * NP,
    )(*o_panels, dpart, b_prev, w)


def _head_body(*refs):
    os_ = refs[:NP]
    (d_ref, b_ref, batch_ref, wl1_ref, bl1_ref, wl2_ref, bl2_ref,
     out_ref, pooled_acc, cnt_acc) = refs[NP:]
    i = pl.program_id(0)

    @pl.when(i == 0)
    def _():
        pooled_acc[...] = jnp.zeros_like(pooled_acc)
        cnt_acc[...] = jnp.zeros_like(cnt_acc)
        out_ref[...] = jnp.zeros_like(out_ref)

    dinv = _dinv_of(d_ref)
    h = _relu_concat(os_, dinv, b_ref)
    seg = lax.broadcasted_iota(jnp.int32, (G, _NB), 0)
    mask = (seg == batch_ref[0, 0, :][None, :]).astype(jnp.float32)
    pooled_acc[...] += jnp.dot(mask, h, preferred_element_type=jnp.float32)
    cnt_acc[...] += jnp.broadcast_to(
        jnp.sum(mask, axis=1, keepdims=True), (G, 128))

    @pl.when(i == pl.num_programs(0) - 1)
    def _():
        cnt = cnt_acc[:, 0:1]
        pooled = pooled_acc[...] / jnp.maximum(cnt, 1.0)
        z = jnp.maximum(
            jnp.dot(pooled, wl1_ref[...], preferred_element_type=jnp.float32)
            + bl1_ref[...], 0.0)
        out_ref[...] = (
            jnp.dot(z, wl2_ref[...], preferred_element_type=jnp.float32)
            + bl2_ref[...])


def _head(o_panels, dpart, b3, batch3d, wl1, bl1, wl2p, bl2p):
    return pl.pallas_call(
        _head_body,
        grid=(N // _NB,),
        in_specs=[pl.BlockSpec((_NB, PW), lambda i: (i, 0))] * NP + [
            pl.BlockSpec((2, _NB, 8), lambda i: (0, i, 0)),
            pl.BlockSpec((1, H), lambda i: (0, 0)),
            pl.BlockSpec((1, 1, _NB), lambda i: (i, 0, 0)),
            pl.BlockSpec((H, H // 2), lambda i: (0, 0)),
            pl.BlockSpec((1, H // 2), lambda i: (0, 0)),
            pl.BlockSpec((H // 2, 128), lambda i: (0, 0)),
            pl.BlockSpec((1, 128), lambda i: (0, 0)),
        ],
        out_specs=pl.BlockSpec((G, 128), lambda i: (0, 0)),
        out_shape=jax.ShapeDtypeStruct((G, 128), jnp.float32),
        scratch_shapes=[
            pltpu.VMEM((G, H), jnp.float32),
            pltpu.VMEM((G, 128), jnp.float32),
        ],
    )(*o_panels, dpart, b3, batch3d, wl1, bl1, wl2p, bl2p)


def kernel(x, edge_index, batch, W1, b1, W2, b2, W3, b3, Wl1, bl1, Wl2, bl2):
    pad = EPAD - E
    src_pad = jnp.concatenate(
        [edge_index[0], jnp.zeros((pad,), jnp.int32)]).reshape(16, CHUNKS, B)
    dst_pad = jnp.concatenate(
        [edge_index[1], jnp.full((pad,), N, jnp.int32)]).reshape(16, CHUNKS, B)
    ones_hbm = jnp.ones((NACC, 8), jnp.float32)

    dpart = _deg_sc(dst_pad, ones_hbm)

    u = _mm_first(x, W1, dpart)
    o = _agg_sc(u, src_pad, dst_pad)
    u = _mm_mid(o, dpart, b1.reshape(1, H), W2)
    o = _agg_sc(u, src_pad, dst_pad)
    u = _mm_mid(o, dpart, b2.reshape(1, H), W3)
    o = _agg_sc(u, src_pad, dst_pad)

    out = _head(o, dpart, b3.reshape(1, H),
                batch.reshape(N // _NB, 1, _NB),
                Wl1, bl1.reshape(1, H // 2),
                jnp.pad(Wl2, ((0, 0), (0, 127))),
                jnp.broadcast_to(bl2.reshape(1, 1), (1, 128)))
    return out[:, 0]
